# baseline (speedup 1.0000x reference)
import functools

import numpy as np
import jax
import jax.numpy as jnp
from jax import lax
from jax.experimental import pallas as pl
from jax.experimental.pallas import tpu as pltpu
from jax.experimental.pallas import tpu_sc as plsc

_ND = 10000
_NH, _HD, _F = 8, 16, 128
_FE = 144
_NRELS = 5
_NC, _NS = 2, 16
_K = 64
_NACC = 10240
_TRASH = 10000
_G = 4
_EPSUB = (10240, 10240, 5120, 10240, 10240)
_NCH = tuple(p // _K for p in _EPSUB)
_NGRP = tuple(n // _G for n in _NCH)
_CHOFF = (0, 7680, 15360, 19200, 26880)
_CORE_RELS = ((0, 1, 2), (3, 4))
_ZROWS = _NACC // _NS

_SIGMA = np.array([h * 16 + d for d in range(16) for h in range(8)], np.int32)


def _stage1(Hs, Hd, Wp, ALp, ARp):
    B = 2000

    def body(hs, hd, w, alp, arp, fs, el, er):
        fsb = jnp.dot(hs[0], w[0], preferred_element_type=jnp.float32)
        fs[0] = fsb
        el[0] = jnp.dot(fsb, alp[0], preferred_element_type=jnp.float32)
        fdb = jnp.dot(hd[0], w[0], preferred_element_type=jnp.float32)
        er[0] = jnp.dot(fdb, arp[0], preferred_element_type=jnp.float32)

    return pl.pallas_call(
        body,
        grid=(_NRELS, _ND // B),
        in_specs=[
            pl.BlockSpec((1, B, _F), lambda r, i: (r, i, 0)),
            pl.BlockSpec((1, B, _F), lambda r, i: (r, i, 0)),
            pl.BlockSpec((1, _F, _F), lambda r, i: (r, 0, 0)),
            pl.BlockSpec((1, _F, 16), lambda r, i: (r, 0, 0)),
            pl.BlockSpec((1, _F, 16), lambda r, i: (r, 0, 0)),
        ],
        out_specs=[
            pl.BlockSpec((1, B, _F), lambda r, i: (r, i, 0)),
            pl.BlockSpec((1, B, 16), lambda r, i: (r, i, 0)),
            pl.BlockSpec((1, B, 16), lambda r, i: (r, i, 0)),
        ],
        out_shape=[
            jax.ShapeDtypeStruct((_NRELS, _ND, _F), jnp.float32),
            jax.ShapeDtypeStruct((_NRELS, _ND, 16), jnp.float32),
            jax.ShapeDtypeStruct((_NRELS, _ND, 16), jnp.float32),
        ],
    )(Hs, Hd, Wp, ALp, ARp)


def _sc_edge_pass(fsel_tbl, er_tbl, comb, zrow):
    mesh = plsc.VectorSubcoreMesh(core_axis_name="c", subcore_axis_name="s")

    @functools.partial(
        pl.kernel,
        out_type=jax.ShapeDtypeStruct((_NRELS, _NACC, _FE), jnp.float32),
        mesh=mesh,
        compiler_params=pltpu.CompilerParams(use_tc_tiling_on_sc=False),
        scratch_types=[
            pltpu.VMEM_SHARED((_NACC, _FE), jnp.float32),
            pltpu.VMEM((3 * _G, _K), jnp.int32),
            pltpu.VMEM((3 * _G, _K), jnp.int32),
            pltpu.VMEM((_K, _FE), jnp.float32),
            pltpu.VMEM((_K, _FE), jnp.float32),
            pltpu.VMEM((_K, 16), jnp.float32),
            pltpu.VMEM((_K, 16), jnp.float32),
            pltpu.SemaphoreType.DMA,
            pltpu.SemaphoreType.DMA,
        ],
    )
    def k(fs_h, er_h, comb_h, zrow_h,
          acc_o, acc_a, idxA, idxB, rows0, rows1, er0, er1, sg0, sg1):
        cid = lax.axis_index("c")
        sid = lax.axis_index("s")
        z0 = sid * _ZROWS
        dbufs = ((rows0, er0, sg0), (rows1, er1, sg1))

        for c in range(_NC):
            @pl.when(cid == c)
            def _():
                for r in _CORE_RELS[c]:
                    nch = _NCH[r]
                    ngrp = _NGRP[r]

                    def firegrp(g, gidx):
                        rowbase = _CHOFF[r] + (sid * nch + g * _G) * 3
                        pltpu.sync_copy(comb_h.at[pl.ds(rowbase, 3 * _G)],
                                        gidx)

                    def fire(t, gidx, b):
                        rows, erb, sg = dbufs[b]
                        pltpu.async_copy(fs_h.at[gidx.at[3 * t]], rows, sg)
                        pltpu.async_copy(er_h.at[gidx.at[3 * t + 1]], erb, sg)

                    def work(t, gidx, b):
                        rows, erb, sg = dbufs[b]
                        pltpu.make_async_copy(
                            fs_h.at[gidx.at[3 * t]], rows, sg).wait()
                        pltpu.make_async_copy(
                            er_h.at[gidx.at[3 * t + 1]], erb, sg).wait()

                        @plsc.parallel_loop(0, _K, unroll=2)
                        def _edge(i):
                            sle = pl.ds(_F, 16)
                            x = rows[i, sle] + erb[i]
                            ee = jnp.exp(jnp.maximum(x, x * 0.2))
                            rows[i, sle] = ee
                            for jj in range(_NH):
                                sl = pl.ds(jj * 16, 16)
                                rows[i, sl] = rows[i, sl] * ee

                        pltpu.sync_copy(rows, acc_a.at[gidx.at[3 * t + 2]],
                                        add=True)

                    def gstage(g, cur, nxt):
                        for t in range(_G - 1):
                            fire(t + 1, cur, (t + 1) % 2)
                            work(t, cur, t % 2)

                        @pl.when(g + 1 < ngrp)
                        def _pf():
                            firegrp(g + 1, nxt)
                            fire(0, nxt, 0)

                        work(_G - 1, cur, (_G - 1) % 2)

                    pltpu.sync_copy(zrow_h, acc_a.at[pl.ds(z0, _ZROWS)])
                    plsc.subcore_barrier()

                    firegrp(0, idxA)
                    fire(0, idxA, 0)

                    @pl.loop(0, ngrp, step=2)
                    def _grp(g):
                        gstage(g, idxA, idxB)
                        gstage(g + 1, idxB, idxA)

                    plsc.subcore_barrier()
                    pltpu.sync_copy(acc_a.at[pl.ds(z0, _ZROWS)],
                                    acc_o.at[r, pl.ds(z0, _ZROWS)])
                    plsc.subcore_barrier()

    return k(fsel_tbl, er_tbl, comb, zrow)


def _semantic_call(numr, denr, bp, W1p, b1v, w2v, PT, R):
    B = 1000

    def body(num, den, bpr, w1, b1r, w2r, pt, out):
        w1v = w1[...]
        ptv = pt[...]
        feats = []
        scores = []
        for r in range(R):
            d = jnp.maximum(den[r], 1e-9)
            d128 = jnp.concatenate([d] * 8, axis=-1)
            f = num[r] / d128 + bpr[r][None, :]
            feats.append(f)
            x = jnp.tanh(jnp.dot(f, w1v, preferred_element_type=jnp.float32)
                         + b1r[0][None, :])
            scores.append(jnp.sum(x * w2r[0][None, :], axis=1, keepdims=True))
        m = scores[0]
        for r in range(1, R):
            m = jnp.maximum(m, scores[r])
        es = [jnp.exp(s - m) for s in scores]
        tot = es[0]
        for r in range(1, R):
            tot = tot + es[r]
        acc = feats[0] * (es[0] / tot)
        for r in range(1, R):
            acc = acc + feats[r] * (es[r] / tot)
        out[...] = jnp.dot(acc, ptv, preferred_element_type=jnp.float32)

    return pl.pallas_call(
        body,
        grid=(_ND // B,),
        in_specs=[
            pl.BlockSpec((R, B, _F), lambda i: (0, i, 0)),
            pl.BlockSpec((R, B, 16), lambda i: (0, i, 0)),
            pl.BlockSpec((R, _F), lambda i: (0, 0)),
            pl.BlockSpec((_F, _F), lambda i: (0, 0)),
            pl.BlockSpec((1, _F), lambda i: (0, 0)),
            pl.BlockSpec((1, _F), lambda i: (0, 0)),
            pl.BlockSpec((_F, _F), lambda i: (0, 0)),
        ],
        out_specs=pl.BlockSpec((B, _F), lambda i: (i, 0)),
        out_shape=jax.ShapeDtypeStruct((_ND, _F), jnp.float32),
    )(numr, denr, bp, W1p, b1v, w2v, PT)


def kernel(h_drug, h_protein, h_sideeffect, ei_d2d, ei_d2p, ei_p2d, ei_p2p,
           ei_s2d, W_d2d, al_d2d, ar_d2d, b_d2d, W_d2p, al_d2p, ar_d2p, b_d2p,
           W_p2d, al_p2d, ar_p2d, b_p2d, W_p2p, al_p2p, ar_p2p, b_p2p,
           W_s2d, al_s2d, ar_s2d, b_s2d, W1, b1, W2, b2):
    f32 = jnp.float32
    sigma = jnp.asarray(_SIGMA)

    hs_pad = jnp.pad(h_sideeffect, ((0, _ND - h_sideeffect.shape[0]), (0, 0)))
    Hs = jnp.stack([h_drug, h_protein, hs_pad, h_drug, h_protein])
    Hd = jnp.stack([h_drug, h_drug, h_drug, h_protein, h_protein])
    Ws = jnp.stack([W_d2d, W_p2d, W_s2d, W_d2p, W_p2p])
    Wp = Ws[:, :, sigma]

    eye_dup = jnp.tile(jnp.eye(_NH, dtype=f32), (1, 2))

    def dup_mat(a):
        return (a.T[:, :, None] * eye_dup[None]).reshape(_F, 16)

    ALp = jnp.stack([dup_mat(al_d2d), dup_mat(al_p2d), dup_mat(al_s2d),
                     dup_mat(al_d2p), dup_mat(al_p2p)])
    ARp = jnp.stack([dup_mat(ar_d2d), dup_mat(ar_p2d), dup_mat(ar_s2d),
                     dup_mat(ar_d2p), dup_mat(ar_p2p)])

    fs_t, el_t, er_t = _stage1(Hs, Hd, Wp, ALp, ARp)
    fsel_tbl = jnp.concatenate([fs_t, el_t], axis=-1).reshape(
        _NRELS * _ND, _FE)
    er_tbl = er_t.reshape(_NRELS * _ND, 16)

    eis = (ei_d2d, ei_p2d, ei_s2d, ei_d2p, ei_p2p)
    combs = []
    for r, ei in enumerate(eis):
        c = ei.shape[1] // _NS
        p = _EPSUB[r]
        nch = _NCH[r]
        s2 = (ei[0] + r * _ND).astype(jnp.int32).reshape(_NS, c)
        d2 = (ei[1] + r * _ND).astype(jnp.int32).reshape(_NS, c)
        a2 = ei[1].astype(jnp.int32).reshape(_NS, c)
        s2 = jnp.pad(s2, ((0, 0), (0, p - c))).reshape(_NS, nch, _K)
        d2 = jnp.pad(d2, ((0, 0), (0, p - c))).reshape(_NS, nch, _K)
        a2 = jnp.pad(a2, ((0, 0), (0, p - c)),
                     constant_values=_TRASH).reshape(_NS, nch, _K)
        combs.append(jnp.stack([s2, d2, a2], axis=2).reshape(-1, _K))
    comb = jnp.concatenate(combs)
    zrow = jnp.zeros((_ZROWS, _FE), f32)

    acc = _sc_edge_pass(fsel_tbl, er_tbl, comb, zrow)
    num = acc[:, :_ND, :_F]
    den = acc[:, :_ND, _F:]

    bs = jnp.stack([b_d2d, b_p2d, b_s2d, b_d2p, b_p2p])
    bp = bs[:, sigma]
    W1p = W1[sigma, :]
    PT = jnp.eye(_F, dtype=f32)[sigma]
    w2v = W2.reshape(1, _F)
    b1v = b1.reshape(1, _F)

    out_d = _semantic_call(num[0:3], den[0:3], bp[0:3],
                           W1p, b1v, w2v, PT, 3)
    out_p = _semantic_call(num[3:5], den[3:5], bp[3:5],
                           W1p, b1v, w2v, PT, 2)
    return (out_d, out_p)


# speedup vs baseline: 1.0291x; 1.0291x over previous
import functools

import numpy as np
import jax
import jax.numpy as jnp
from jax import lax
from jax.experimental import pallas as pl
from jax.experimental.pallas import tpu as pltpu
from jax.experimental.pallas import tpu_sc as plsc

_ND = 10000
_NH, _HD, _F = 8, 16, 128
_FE = 144
_NRELS = 5
_NC, _NS = 2, 16
_K = 64
_NACC = 10240
_TRASH = 10000
_G = 4
_EPSUB = (10240, 10240, 5120, 10240, 10240)
_NCH = tuple(p // _K for p in _EPSUB)
_NGRP = tuple(n // _G for n in _NCH)
_CHOFF = (0, 7680, 15360, 19200, 26880)
_CORE_RELS = ((0, 1, 2), (3, 4))
_ZROWS = _NACC // _NS

_SIGMA = np.array([h * 16 + d for d in range(16) for h in range(8)], np.int32)


def _stage1(Hs, Hd, Wp, ALp, ARp):
    B = 2000

    def body(hs, hd, w, alp, arp, fs, el, er):
        fsb = jnp.dot(hs[0], w[0], preferred_element_type=jnp.float32)
        fs[0] = fsb
        el[0] = jnp.dot(fsb, alp[0], preferred_element_type=jnp.float32)
        fdb = jnp.dot(hd[0], w[0], preferred_element_type=jnp.float32)
        er[0] = jnp.dot(fdb, arp[0], preferred_element_type=jnp.float32)

    return pl.pallas_call(
        body,
        grid=(_NRELS, _ND // B),
        in_specs=[
            pl.BlockSpec((1, B, _F), lambda r, i: (r, i, 0)),
            pl.BlockSpec((1, B, _F), lambda r, i: (r, i, 0)),
            pl.BlockSpec((1, _F, _F), lambda r, i: (r, 0, 0)),
            pl.BlockSpec((1, _F, 16), lambda r, i: (r, 0, 0)),
            pl.BlockSpec((1, _F, 16), lambda r, i: (r, 0, 0)),
        ],
        out_specs=[
            pl.BlockSpec((1, B, _F), lambda r, i: (r, i, 0)),
            pl.BlockSpec((1, B, 16), lambda r, i: (r, i, 0)),
            pl.BlockSpec((1, B, 16), lambda r, i: (r, i, 0)),
        ],
        out_shape=[
            jax.ShapeDtypeStruct((_NRELS, _ND, _F), jnp.float32),
            jax.ShapeDtypeStruct((_NRELS, _ND, 16), jnp.float32),
            jax.ShapeDtypeStruct((_NRELS, _ND, 16), jnp.float32),
        ],
    )(Hs, Hd, Wp, ALp, ARp)


def _sc_edge_pass(fsel_tbl, er_tbl, comb, zrow):
    mesh = plsc.VectorSubcoreMesh(core_axis_name="c", subcore_axis_name="s")

    @functools.partial(
        pl.kernel,
        out_type=jax.ShapeDtypeStruct((_NRELS, _NACC, _FE), jnp.float32),
        mesh=mesh,
        compiler_params=pltpu.CompilerParams(use_tc_tiling_on_sc=False),
        scratch_types=[
            pltpu.VMEM_SHARED((_NACC, _FE), jnp.float32),
            pltpu.VMEM((3 * _G, _K), jnp.int32),
            pltpu.VMEM((3 * _G, _K), jnp.int32),
            pltpu.VMEM((_K, _FE), jnp.float32),
            pltpu.VMEM((_K, _FE), jnp.float32),
            pltpu.VMEM((_K, 16), jnp.float32),
            pltpu.VMEM((_K, 16), jnp.float32),
            pltpu.SemaphoreType.DMA,
            pltpu.SemaphoreType.DMA,
            pltpu.SemaphoreType.DMA,
        ],
    )
    def k(fs_h, er_h, comb_h, zrow_h,
          acc_o, acc_a, idxA, idxB, rows0, rows1, er0, er1, sg0, sg1, si):
        cid = lax.axis_index("c")
        sid = lax.axis_index("s")
        z0 = sid * _ZROWS
        dbufs = ((rows0, er0, sg0), (rows1, er1, sg1))

        for c in range(_NC):
            @pl.when(cid == c)
            def _():
                for r in _CORE_RELS[c]:
                    nch = _NCH[r]
                    ngrp = _NGRP[r]

                    def grpbase(g):
                        return _CHOFF[r] + (sid * nch + g * _G) * 3

                    def firegrp(g, gidx):
                        pltpu.async_copy(
                            comb_h.at[pl.ds(grpbase(g), 3 * _G)], gidx, si)

                    def waitgrp(g, gidx):
                        pltpu.make_async_copy(
                            comb_h.at[pl.ds(grpbase(g), 3 * _G)], gidx,
                            si).wait()

                    def fire(t, gidx, b):
                        rows, erb, sg = dbufs[b]
                        pltpu.async_copy(fs_h.at[gidx.at[3 * t]], rows, sg)
                        pltpu.async_copy(er_h.at[gidx.at[3 * t + 1]], erb, sg)

                    def work(t, gidx, b):
                        rows, erb, sg = dbufs[b]
                        pltpu.make_async_copy(
                            fs_h.at[gidx.at[3 * t]], rows, sg).wait()
                        pltpu.make_async_copy(
                            er_h.at[gidx.at[3 * t + 1]], erb, sg).wait()

                        @plsc.parallel_loop(0, _K, unroll=2)
                        def _edge(i):
                            sle = pl.ds(_F, 16)
                            x = rows[i, sle] + erb[i]
                            ee = jnp.exp(jnp.maximum(x, x * 0.2))
                            rows[i, sle] = ee
                            for jj in range(_NH):
                                sl = pl.ds(jj * 16, 16)
                                rows[i, sl] = rows[i, sl] * ee

                        pltpu.sync_copy(rows, acc_a.at[gidx.at[3 * t + 2]],
                                        add=True)

                    def gstage(g, cur, nxt):
                        @pl.when(g + 1 < ngrp)
                        def _pfi():
                            firegrp(g + 1, nxt)

                        for t in range(_G - 1):
                            fire(t + 1, cur, (t + 1) % 2)
                            work(t, cur, t % 2)

                        @pl.when(g + 1 < ngrp)
                        def _pf():
                            waitgrp(g + 1, nxt)
                            fire(0, nxt, 0)

                        work(_G - 1, cur, (_G - 1) % 2)

                    pltpu.sync_copy(zrow_h, acc_a.at[pl.ds(z0, _ZROWS)])
                    plsc.subcore_barrier()

                    firegrp(0, idxA)
                    waitgrp(0, idxA)
                    fire(0, idxA, 0)

                    @pl.loop(0, ngrp, step=2)
                    def _grp(g):
                        gstage(g, idxA, idxB)
                        gstage(g + 1, idxB, idxA)

                    plsc.subcore_barrier()
                    pltpu.sync_copy(acc_a.at[pl.ds(z0, _ZROWS)],
                                    acc_o.at[r, pl.ds(z0, _ZROWS)])
                    plsc.subcore_barrier()

    return k(fsel_tbl, er_tbl, comb, zrow)


def _semantic_call(numr, denr, bp, W1p, b1v, w2v, PT, R):
    B = 1000

    def body(num, den, bpr, w1, b1r, w2r, pt, out):
        w1v = w1[...]
        ptv = pt[...]
        feats = []
        scores = []
        for r in range(R):
            d = jnp.maximum(den[r], 1e-9)
            d128 = jnp.concatenate([d] * 8, axis=-1)
            f = num[r] / d128 + bpr[r][None, :]
            feats.append(f)
            x = jnp.tanh(jnp.dot(f, w1v, preferred_element_type=jnp.float32)
                         + b1r[0][None, :])
            scores.append(jnp.sum(x * w2r[0][None, :], axis=1, keepdims=True))
        m = scores[0]
        for r in range(1, R):
            m = jnp.maximum(m, scores[r])
        es = [jnp.exp(s - m) for s in scores]
        tot = es[0]
        for r in range(1, R):
            tot = tot + es[r]
        acc = feats[0] * (es[0] / tot)
        for r in range(1, R):
            acc = acc + feats[r] * (es[r] / tot)
        out[...] = jnp.dot(acc, ptv, preferred_element_type=jnp.float32)

    return pl.pallas_call(
        body,
        grid=(_ND // B,),
        in_specs=[
            pl.BlockSpec((R, B, _F), lambda i: (0, i, 0)),
            pl.BlockSpec((R, B, 16), lambda i: (0, i, 0)),
            pl.BlockSpec((R, _F), lambda i: (0, 0)),
            pl.BlockSpec((_F, _F), lambda i: (0, 0)),
            pl.BlockSpec((1, _F), lambda i: (0, 0)),
            pl.BlockSpec((1, _F), lambda i: (0, 0)),
            pl.BlockSpec((_F, _F), lambda i: (0, 0)),
        ],
        out_specs=pl.BlockSpec((B, _F), lambda i: (i, 0)),
        out_shape=jax.ShapeDtypeStruct((_ND, _F), jnp.float32),
    )(numr, denr, bp, W1p, b1v, w2v, PT)


def kernel(h_drug, h_protein, h_sideeffect, ei_d2d, ei_d2p, ei_p2d, ei_p2p,
           ei_s2d, W_d2d, al_d2d, ar_d2d, b_d2d, W_d2p, al_d2p, ar_d2p, b_d2p,
           W_p2d, al_p2d, ar_p2d, b_p2d, W_p2p, al_p2p, ar_p2p, b_p2p,
           W_s2d, al_s2d, ar_s2d, b_s2d, W1, b1, W2, b2):
    f32 = jnp.float32
    sigma = jnp.asarray(_SIGMA)

    hs_pad = jnp.pad(h_sideeffect, ((0, _ND - h_sideeffect.shape[0]), (0, 0)))
    Hs = jnp.stack([h_drug, h_protein, hs_pad, h_drug, h_protein])
    Hd = jnp.stack([h_drug, h_drug, h_drug, h_protein, h_protein])
    Ws = jnp.stack([W_d2d, W_p2d, W_s2d, W_d2p, W_p2p])
    Wp = Ws[:, :, sigma]

    eye_dup = jnp.tile(jnp.eye(_NH, dtype=f32), (1, 2))

    def dup_mat(a):
        return (a.T[:, :, None] * eye_dup[None]).reshape(_F, 16)

    ALp = jnp.stack([dup_mat(al_d2d), dup_mat(al_p2d), dup_mat(al_s2d),
                     dup_mat(al_d2p), dup_mat(al_p2p)])
    ARp = jnp.stack([dup_mat(ar_d2d), dup_mat(ar_p2d), dup_mat(ar_s2d),
                     dup_mat(ar_d2p), dup_mat(ar_p2p)])

    fs_t, el_t, er_t = _stage1(Hs, Hd, Wp, ALp, ARp)
    fsel_tbl = jnp.concatenate([fs_t, el_t], axis=-1).reshape(
        _NRELS * _ND, _FE)
    er_tbl = er_t.reshape(_NRELS * _ND, 16)

    eis = (ei_d2d, ei_p2d, ei_s2d, ei_d2p, ei_p2p)
    combs = []
    for r, ei in enumerate(eis):
        c = ei.shape[1] // _NS
        p = _EPSUB[r]
        nch = _NCH[r]
        s2 = (ei[0] + r * _ND).astype(jnp.int32).reshape(_NS, c)
        d2 = (ei[1] + r * _ND).astype(jnp.int32).reshape(_NS, c)
        a2 = ei[1].astype(jnp.int32).reshape(_NS, c)
        s2 = jnp.pad(s2, ((0, 0), (0, p - c))).reshape(_NS, nch, _K)
        d2 = jnp.pad(d2, ((0, 0), (0, p - c))).reshape(_NS, nch, _K)
        a2 = jnp.pad(a2, ((0, 0), (0, p - c)),
                     constant_values=_TRASH).reshape(_NS, nch, _K)
        combs.append(jnp.stack([s2, d2, a2], axis=2).reshape(-1, _K))
    comb = jnp.concatenate(combs)
    zrow = jnp.zeros((_ZROWS, _FE), f32)

    acc = _sc_edge_pass(fsel_tbl, er_tbl, comb, zrow)
    num = acc[:, :_ND, :_F]
    den = acc[:, :_ND, _F:]

    bs = jnp.stack([b_d2d, b_p2d, b_s2d, b_d2p, b_p2p])
    bp = bs[:, sigma]
    W1p = W1[sigma, :]
    PT = jnp.eye(_F, dtype=f32)[sigma]
    w2v = W2.reshape(1, _F)
    b1v = b1.reshape(1, _F)

    out_d = _semantic_call(num[0:3], den[0:3], bp[0:3],
                           W1p, b1v, w2v, PT, 3)
    out_p = _semantic_call(num[3:5], den[3:5], bp[3:5],
                           W1p, b1v, w2v, PT, 2)
    return (out_d, out_p)


# speedup vs baseline: 1.2548x; 1.2194x over previous
import functools

import numpy as np
import jax
import jax.numpy as jnp
from jax import lax
from jax.experimental import pallas as pl
from jax.experimental.pallas import tpu as pltpu
from jax.experimental.pallas import tpu_sc as plsc

_ND = 10000
_NH, _HD, _F = 8, 16, 128
_FE = 144
_NRELS = 5
_NC, _NS = 2, 16
_K = 64
_NACC = 10240
_TRASH = 10000
_EPSUB = (10112, 10112, 5120, 10112, 10112)
_NCH = tuple(p // _K for p in _EPSUB)
_CHOFF = (0, 7584, 15168, 19008, 26592)
_CORE_RELS = ((0, 1, 2), (3, 4))
_ZROWS = _NACC // _NS

_SIGMA = np.array([h * 16 + d for d in range(16) for h in range(8)], np.int32)


def _stage1(Hs, Hd, Wp, ALp, ARp):
    B = 2000

    def body(hs, hd, w, alp, arp, fs, el, er):
        fsb = jnp.dot(hs[0], w[0], preferred_element_type=jnp.float32)
        fs[0] = fsb
        el[0] = jnp.dot(fsb, alp[0], preferred_element_type=jnp.float32)
        fdb = jnp.dot(hd[0], w[0], preferred_element_type=jnp.float32)
        er[0] = jnp.dot(fdb, arp[0], preferred_element_type=jnp.float32)

    return pl.pallas_call(
        body,
        grid=(_NRELS, _ND // B),
        in_specs=[
            pl.BlockSpec((1, B, _F), lambda r, i: (r, i, 0)),
            pl.BlockSpec((1, B, _F), lambda r, i: (r, i, 0)),
            pl.BlockSpec((1, _F, _F), lambda r, i: (r, 0, 0)),
            pl.BlockSpec((1, _F, 16), lambda r, i: (r, 0, 0)),
            pl.BlockSpec((1, _F, 16), lambda r, i: (r, 0, 0)),
        ],
        out_specs=[
            pl.BlockSpec((1, B, _F), lambda r, i: (r, i, 0)),
            pl.BlockSpec((1, B, 16), lambda r, i: (r, i, 0)),
            pl.BlockSpec((1, B, 16), lambda r, i: (r, i, 0)),
        ],
        out_shape=[
            jax.ShapeDtypeStruct((_NRELS, _ND, _F), jnp.float32),
            jax.ShapeDtypeStruct((_NRELS, _ND, 16), jnp.float32),
            jax.ShapeDtypeStruct((_NRELS, _ND, 16), jnp.float32),
        ],
    )(Hs, Hd, Wp, ALp, ARp)


def _sc_edge_pass(fs_tbl, el_tbl, er_tbl, comb, zrow, zden):
    mesh = plsc.VectorSubcoreMesh(core_axis_name="c", subcore_axis_name="s")

    @functools.partial(
        pl.kernel,
        out_type=(
            jax.ShapeDtypeStruct((_NRELS, _NACC, _F), jnp.float32),
            jax.ShapeDtypeStruct((_NRELS, _NACC, 16), jnp.float32),
        ),
        mesh=mesh,
        compiler_params=pltpu.CompilerParams(use_tc_tiling_on_sc=False),
        scratch_types=[
            pltpu.VMEM_SHARED((_NACC, _F), jnp.float32),
            pltpu.VMEM_SHARED((_NACC, 16), jnp.float32),
            pltpu.VMEM((3, _K), jnp.int32),
            pltpu.VMEM((3, _K), jnp.int32),
            pltpu.VMEM((_K, _F), jnp.float32),
            pltpu.VMEM((_K, _F), jnp.float32),
            pltpu.VMEM((_K, 16), jnp.float32),
            pltpu.VMEM((_K, 16), jnp.float32),
            pltpu.VMEM((_K, 16), jnp.float32),
            pltpu.VMEM((_K, 16), jnp.float32),
            pltpu.SemaphoreType.DMA,
            pltpu.SemaphoreType.DMA,
        ],
    )
    def k(fs_h, el_h, er_h, comb_h, zrow_h, zden_h,
          num_o, den_o, num_a, den_a, idx0, idx1, rows0, rows1,
          el0, el1, er0, er1, sg0, sg1):
        cid = lax.axis_index("c")
        sid = lax.axis_index("s")
        z0 = sid * _ZROWS
        bufs = ((idx0, rows0, el0, er0, sg0), (idx1, rows1, el1, er1, sg1))

        for c in range(_NC):
            @pl.when(cid == c)
            def _():
                for r in _CORE_RELS[c]:
                    nch = _NCH[r]

                    def fire(j, b):
                        idxv, rows, elb, erb, sg = bufs[b]
                        rowbase = _CHOFF[r] + (sid * nch + j) * 3
                        pltpu.sync_copy(comb_h.at[pl.ds(rowbase, 3)], idxv)
                        pltpu.async_copy(fs_h.at[idxv.at[0]], rows, sg)
                        pltpu.async_copy(el_h.at[idxv.at[0]], elb, sg)
                        pltpu.async_copy(er_h.at[idxv.at[1]], erb, sg)

                    def stage(j, b):
                        idxv, rows, elb, erb, sg = bufs[b]

                        @pl.when(j + 1 < nch)
                        def _pf():
                            fire(j + 1, 1 - b)

                        pltpu.make_async_copy(fs_h.at[idxv.at[0]], rows, sg).wait()
                        pltpu.make_async_copy(el_h.at[idxv.at[0]], elb, sg).wait()
                        pltpu.make_async_copy(er_h.at[idxv.at[1]], erb, sg).wait()

                        @plsc.parallel_loop(0, _K, unroll=4)
                        def _edge(i):
                            x = elb[i] + erb[i]
                            ee = jnp.exp(jnp.maximum(x, x * 0.2))
                            elb[i] = ee
                            for jj in range(_NH):
                                sl = pl.ds(jj * 16, 16)
                                rows[i, sl] = rows[i, sl] * ee

                        pltpu.sync_copy(rows, num_a.at[idxv.at[2]], add=True)
                        pltpu.sync_copy(elb, den_a.at[idxv.at[2]], add=True)

                    pltpu.sync_copy(zrow_h, num_a.at[pl.ds(z0, _ZROWS)])
                    pltpu.sync_copy(zden_h, den_a.at[pl.ds(z0, _ZROWS)])
                    plsc.subcore_barrier()

                    fire(0, 0)

                    @pl.loop(0, nch, step=2)
                    def _chunk(kk):
                        stage(kk, 0)
                        stage(kk + 1, 1)

                    plsc.subcore_barrier()
                    pltpu.sync_copy(num_a.at[pl.ds(z0, _ZROWS)],
                                    num_o.at[r, pl.ds(z0, _ZROWS)])
                    pltpu.sync_copy(den_a.at[pl.ds(z0, _ZROWS)],
                                    den_o.at[r, pl.ds(z0, _ZROWS)])
                    plsc.subcore_barrier()

    return k(fs_tbl, el_tbl, er_tbl, comb, zrow, zden)


def _semantic_call(numr, denr, bp, W1p, b1v, w2v, PT, R):
    B = 1000

    def body(num, den, bpr, w1, b1r, w2r, pt, out):
        w1v = w1[...]
        ptv = pt[...]
        feats = []
        scores = []
        for r in range(R):
            d = jnp.maximum(den[r], 1e-9)
            d128 = jnp.concatenate([d] * 8, axis=-1)
            f = num[r] / d128 + bpr[r][None, :]
            feats.append(f)
            x = jnp.tanh(jnp.dot(f, w1v, preferred_element_type=jnp.float32)
                         + b1r[0][None, :])
            scores.append(jnp.sum(x * w2r[0][None, :], axis=1, keepdims=True))
        m = scores[0]
        for r in range(1, R):
            m = jnp.maximum(m, scores[r])
        es = [jnp.exp(s - m) for s in scores]
        tot = es[0]
        for r in range(1, R):
            tot = tot + es[r]
        acc = feats[0] * (es[0] / tot)
        for r in range(1, R):
            acc = acc + feats[r] * (es[r] / tot)
        out[...] = jnp.dot(acc, ptv, preferred_element_type=jnp.float32)

    return pl.pallas_call(
        body,
        grid=(_ND // B,),
        in_specs=[
            pl.BlockSpec((R, B, _F), lambda i: (0, i, 0)),
            pl.BlockSpec((R, B, 16), lambda i: (0, i, 0)),
            pl.BlockSpec((R, _F), lambda i: (0, 0)),
            pl.BlockSpec((_F, _F), lambda i: (0, 0)),
            pl.BlockSpec((1, _F), lambda i: (0, 0)),
            pl.BlockSpec((1, _F), lambda i: (0, 0)),
            pl.BlockSpec((_F, _F), lambda i: (0, 0)),
        ],
        out_specs=pl.BlockSpec((B, _F), lambda i: (i, 0)),
        out_shape=jax.ShapeDtypeStruct((_ND, _F), jnp.float32),
    )(numr, denr, bp, W1p, b1v, w2v, PT)


def kernel(h_drug, h_protein, h_sideeffect, ei_d2d, ei_d2p, ei_p2d, ei_p2p,
           ei_s2d, W_d2d, al_d2d, ar_d2d, b_d2d, W_d2p, al_d2p, ar_d2p, b_d2p,
           W_p2d, al_p2d, ar_p2d, b_p2d, W_p2p, al_p2p, ar_p2p, b_p2p,
           W_s2d, al_s2d, ar_s2d, b_s2d, W1, b1, W2, b2):
    f32 = jnp.float32
    sigma = jnp.asarray(_SIGMA)

    hs_pad = jnp.pad(h_sideeffect, ((0, _ND - h_sideeffect.shape[0]), (0, 0)))
    Hs = jnp.stack([h_drug, h_protein, hs_pad, h_drug, h_protein])
    Hd = jnp.stack([h_drug, h_drug, h_drug, h_protein, h_protein])
    Ws = jnp.stack([W_d2d, W_p2d, W_s2d, W_d2p, W_p2p])
    Wp = Ws[:, :, sigma]

    eye_dup = jnp.tile(jnp.eye(_NH, dtype=f32), (1, 2))

    def dup_mat(a):
        return (a.T[:, :, None] * eye_dup[None]).reshape(_F, 16)

    ALp = jnp.stack([dup_mat(al_d2d), dup_mat(al_p2d), dup_mat(al_s2d),
                     dup_mat(al_d2p), dup_mat(al_p2p)])
    ARp = jnp.stack([dup_mat(ar_d2d), dup_mat(ar_p2d), dup_mat(ar_s2d),
                     dup_mat(ar_d2p), dup_mat(ar_p2p)])

    fs_t, el_t, er_t = _stage1(Hs, Hd, Wp, ALp, ARp)
    fs_tbl = fs_t.reshape(_NRELS * _ND, _F)
    el_tbl = el_t.reshape(_NRELS * _ND, 16)
    er_tbl = er_t.reshape(_NRELS * _ND, 16)

    eis = (ei_d2d, ei_p2d, ei_s2d, ei_d2p, ei_p2p)
    combs = []
    for r, ei in enumerate(eis):
        c = ei.shape[1] // _NS
        p = _EPSUB[r]
        nch = _NCH[r]
        s2 = (ei[0] + r * _ND).astype(jnp.int32).reshape(_NS, c)
        d2 = (ei[1] + r * _ND).astype(jnp.int32).reshape(_NS, c)
        a2 = ei[1].astype(jnp.int32).reshape(_NS, c)
        s2 = jnp.pad(s2, ((0, 0), (0, p - c))).reshape(_NS, nch, _K)
        d2 = jnp.pad(d2, ((0, 0), (0, p - c))).reshape(_NS, nch, _K)
        a2 = jnp.pad(a2, ((0, 0), (0, p - c)),
                     constant_values=_TRASH).reshape(_NS, nch, _K)
        combs.append(jnp.stack([s2, d2, a2], axis=2).reshape(-1, _K))
    comb = jnp.concatenate(combs)
    zrow = jnp.zeros((_ZROWS, _F), f32)
    zden = jnp.zeros((_ZROWS, 16), f32)

    num, den = _sc_edge_pass(fs_tbl, el_tbl, er_tbl, comb, zrow, zden)
    num = num[:, :_ND]
    den = den[:, :_ND]

    bs = jnp.stack([b_d2d, b_p2d, b_s2d, b_d2p, b_p2p])
    bp = bs[:, sigma]
    W1p = W1[sigma, :]
    PT = jnp.eye(_F, dtype=f32)[sigma]
    w2v = W2.reshape(1, _F)
    b1v = b1.reshape(1, _F)

    out_d = _semantic_call(num[0:3], den[0:3], bp[0:3],
                           W1p, b1v, w2v, PT, 3)
    out_p = _semantic_call(num[3:5], den[3:5], bp[3:5],
                           W1p, b1v, w2v, PT, 2)
    return (out_d, out_p)


# speedup vs baseline: 1.2734x; 1.0148x over previous
import functools

import numpy as np
import jax
import jax.numpy as jnp
from jax import lax
from jax.experimental import pallas as pl
from jax.experimental.pallas import tpu as pltpu
from jax.experimental.pallas import tpu_sc as plsc

_ND = 10000
_NH, _HD, _F = 8, 16, 128
_FE = 144
_NRELS = 5
_NC, _NS = 2, 16
_K = 64
_NACC = 10240
_TRASH = 10000
_NSLOTS = 6
_SLOT_REL = (0, 1, 2, 3, 4, 2)
_EPSUB = (10112, 10112, 2560, 10112, 10112, 2560)
_NCH = tuple(p // _K for p in _EPSUB)
_CHOFF = (0, 7584, 15168, 17088, 24672, 32256)
_CORE_RELS = ((0, 1, 2), (3, 4, 5))
_ZROWS = _NACC // _NS

_SIGMA = np.array([h * 16 + d for d in range(16) for h in range(8)], np.int32)


def _stage1(Hs, Hd, Wp, ALp, ARp):
    B = 2000

    def body(hs, hd, w, alp, arp, fs, el, er):
        fsb = jnp.dot(hs[0], w[0], preferred_element_type=jnp.float32)
        fs[0] = fsb
        el[0] = jnp.dot(fsb, alp[0], preferred_element_type=jnp.float32)
        fdb = jnp.dot(hd[0], w[0], preferred_element_type=jnp.float32)
        er[0] = jnp.dot(fdb, arp[0], preferred_element_type=jnp.float32)

    return pl.pallas_call(
        body,
        grid=(_NRELS, _ND // B),
        in_specs=[
            pl.BlockSpec((1, B, _F), lambda r, i: (r, i, 0)),
            pl.BlockSpec((1, B, _F), lambda r, i: (r, i, 0)),
            pl.BlockSpec((1, _F, _F), lambda r, i: (r, 0, 0)),
            pl.BlockSpec((1, _F, 16), lambda r, i: (r, 0, 0)),
            pl.BlockSpec((1, _F, 16), lambda r, i: (r, 0, 0)),
        ],
        out_specs=[
            pl.BlockSpec((1, B, _F), lambda r, i: (r, i, 0)),
            pl.BlockSpec((1, B, 16), lambda r, i: (r, i, 0)),
            pl.BlockSpec((1, B, 16), lambda r, i: (r, i, 0)),
        ],
        out_shape=[
            jax.ShapeDtypeStruct((_NRELS, _ND, _F), jnp.float32),
            jax.ShapeDtypeStruct((_NRELS, _ND, 16), jnp.float32),
            jax.ShapeDtypeStruct((_NRELS, _ND, 16), jnp.float32),
        ],
    )(Hs, Hd, Wp, ALp, ARp)


def _sc_edge_pass(fs_tbl, el_tbl, er_tbl, comb, zrow, zden):
    mesh = plsc.VectorSubcoreMesh(core_axis_name="c", subcore_axis_name="s")

    @functools.partial(
        pl.kernel,
        out_type=(
            jax.ShapeDtypeStruct((_NSLOTS, _NACC, _F), jnp.float32),
            jax.ShapeDtypeStruct((_NSLOTS, _NACC, 16), jnp.float32),
        ),
        mesh=mesh,
        compiler_params=pltpu.CompilerParams(use_tc_tiling_on_sc=False),
        scratch_types=[
            pltpu.VMEM_SHARED((_NACC, _F), jnp.float32),
            pltpu.VMEM_SHARED((_NACC, 16), jnp.float32),
            pltpu.VMEM((3, _K), jnp.int32),
            pltpu.VMEM((3, _K), jnp.int32),
            pltpu.VMEM((_K, _F), jnp.float32),
            pltpu.VMEM((_K, _F), jnp.float32),
            pltpu.VMEM((_K, 16), jnp.float32),
            pltpu.VMEM((_K, 16), jnp.float32),
            pltpu.VMEM((_K, 16), jnp.float32),
            pltpu.VMEM((_K, 16), jnp.float32),
            pltpu.SemaphoreType.DMA,
            pltpu.SemaphoreType.DMA,
        ],
    )
    def k(fs_h, el_h, er_h, comb_h, zrow_h, zden_h,
          num_o, den_o, num_a, den_a, idx0, idx1, rows0, rows1,
          el0, el1, er0, er1, sg0, sg1):
        cid = lax.axis_index("c")
        sid = lax.axis_index("s")
        z0 = sid * _ZROWS
        bufs = ((idx0, rows0, el0, er0, sg0), (idx1, rows1, el1, er1, sg1))

        for c in range(_NC):
            @pl.when(cid == c)
            def _():
                for r in _CORE_RELS[c]:
                    nch = _NCH[r]

                    def fire(j, b):
                        idxv, rows, elb, erb, sg = bufs[b]
                        rowbase = _CHOFF[r] + (sid * nch + j) * 3
                        pltpu.sync_copy(comb_h.at[pl.ds(rowbase, 3)], idxv)
                        pltpu.async_copy(fs_h.at[idxv.at[0]], rows, sg)
                        pltpu.async_copy(el_h.at[idxv.at[0]], elb, sg)
                        pltpu.async_copy(er_h.at[idxv.at[1]], erb, sg)

                    def stage(j, b):
                        idxv, rows, elb, erb, sg = bufs[b]

                        @pl.when(j + 1 < nch)
                        def _pf():
                            fire(j + 1, 1 - b)

                        pltpu.make_async_copy(fs_h.at[idxv.at[0]], rows, sg).wait()
                        pltpu.make_async_copy(el_h.at[idxv.at[0]], elb, sg).wait()
                        pltpu.make_async_copy(er_h.at[idxv.at[1]], erb, sg).wait()

                        @plsc.parallel_loop(0, _K, unroll=4)
                        def _edge(i):
                            x = elb[i] + erb[i]
                            ee = jnp.exp(jnp.maximum(x, x * 0.2))
                            elb[i] = ee
                            for jj in range(_NH):
                                sl = pl.ds(jj * 16, 16)
                                rows[i, sl] = rows[i, sl] * ee

                        pltpu.sync_copy(rows, num_a.at[idxv.at[2]], add=True)
                        pltpu.sync_copy(elb, den_a.at[idxv.at[2]], add=True)

                    pltpu.sync_copy(zrow_h, num_a.at[pl.ds(z0, _ZROWS)])
                    pltpu.sync_copy(zden_h, den_a.at[pl.ds(z0, _ZROWS)])
                    plsc.subcore_barrier()

                    fire(0, 0)

                    @pl.loop(0, nch, step=2)
                    def _chunk(kk):
                        stage(kk, 0)
                        stage(kk + 1, 1)

                    plsc.subcore_barrier()
                    pltpu.sync_copy(num_a.at[pl.ds(z0, _ZROWS)],
                                    num_o.at[r, pl.ds(z0, _ZROWS)])
                    pltpu.sync_copy(den_a.at[pl.ds(z0, _ZROWS)],
                                    den_o.at[r, pl.ds(z0, _ZROWS)])
                    plsc.subcore_barrier()

    return k(fs_tbl, el_tbl, er_tbl, comb, zrow, zden)


def _semantic_call(numr, denr, bp, W1p, b1v, w2v, PT, R):
    B = 1000

    def body(num, den, bpr, w1, b1r, w2r, pt, out):
        w1v = w1[...]
        ptv = pt[...]
        feats = []
        scores = []
        for r in range(R):
            d = jnp.maximum(den[r], 1e-9)
            d128 = jnp.concatenate([d] * 8, axis=-1)
            f = num[r] / d128 + bpr[r][None, :]
            feats.append(f)
            x = jnp.tanh(jnp.dot(f, w1v, preferred_element_type=jnp.float32)
                         + b1r[0][None, :])
            scores.append(jnp.sum(x * w2r[0][None, :], axis=1, keepdims=True))
        m = scores[0]
        for r in range(1, R):
            m = jnp.maximum(m, scores[r])
        es = [jnp.exp(s - m) for s in scores]
        tot = es[0]
        for r in range(1, R):
            tot = tot + es[r]
        acc = feats[0] * (es[0] / tot)
        for r in range(1, R):
            acc = acc + feats[r] * (es[r] / tot)
        out[...] = jnp.dot(acc, ptv, preferred_element_type=jnp.float32)

    return pl.pallas_call(
        body,
        grid=(_ND // B,),
        in_specs=[
            pl.BlockSpec((R, B, _F), lambda i: (0, i, 0)),
            pl.BlockSpec((R, B, 16), lambda i: (0, i, 0)),
            pl.BlockSpec((R, _F), lambda i: (0, 0)),
            pl.BlockSpec((_F, _F), lambda i: (0, 0)),
            pl.BlockSpec((1, _F), lambda i: (0, 0)),
            pl.BlockSpec((1, _F), lambda i: (0, 0)),
            pl.BlockSpec((_F, _F), lambda i: (0, 0)),
        ],
        out_specs=pl.BlockSpec((B, _F), lambda i: (i, 0)),
        out_shape=jax.ShapeDtypeStruct((_ND, _F), jnp.float32),
    )(numr, denr, bp, W1p, b1v, w2v, PT)


def kernel(h_drug, h_protein, h_sideeffect, ei_d2d, ei_d2p, ei_p2d, ei_p2p,
           ei_s2d, W_d2d, al_d2d, ar_d2d, b_d2d, W_d2p, al_d2p, ar_d2p, b_d2p,
           W_p2d, al_p2d, ar_p2d, b_p2d, W_p2p, al_p2p, ar_p2p, b_p2p,
           W_s2d, al_s2d, ar_s2d, b_s2d, W1, b1, W2, b2):
    f32 = jnp.float32
    sigma = jnp.asarray(_SIGMA)

    hs_pad = jnp.pad(h_sideeffect, ((0, _ND - h_sideeffect.shape[0]), (0, 0)))
    Hs = jnp.stack([h_drug, h_protein, hs_pad, h_drug, h_protein])
    Hd = jnp.stack([h_drug, h_drug, h_drug, h_protein, h_protein])
    Ws = jnp.stack([W_d2d, W_p2d, W_s2d, W_d2p, W_p2p])
    Wp = Ws[:, :, sigma]

    eye_dup = jnp.tile(jnp.eye(_NH, dtype=f32), (1, 2))

    def dup_mat(a):
        return (a.T[:, :, None] * eye_dup[None]).reshape(_F, 16)

    ALp = jnp.stack([dup_mat(al_d2d), dup_mat(al_p2d), dup_mat(al_s2d),
                     dup_mat(al_d2p), dup_mat(al_p2p)])
    ARp = jnp.stack([dup_mat(ar_d2d), dup_mat(ar_p2d), dup_mat(ar_s2d),
                     dup_mat(ar_d2p), dup_mat(ar_p2p)])

    fs_t, el_t, er_t = _stage1(Hs, Hd, Wp, ALp, ARp)
    fs_tbl = fs_t.reshape(_NRELS * _ND, _F)
    el_tbl = el_t.reshape(_NRELS * _ND, 16)
    er_tbl = er_t.reshape(_NRELS * _ND, 16)

    half = ei_s2d.shape[1] // 2
    eis = (ei_d2d, ei_p2d, ei_s2d[:, :half], ei_d2p, ei_p2p, ei_s2d[:, half:])
    combs = []
    for r, ei in enumerate(eis):
        rt = _SLOT_REL[r]
        c = ei.shape[1] // _NS
        p = _EPSUB[r]
        nch = _NCH[r]
        s2 = (ei[0] + rt * _ND).astype(jnp.int32).reshape(_NS, c)
        d2 = (ei[1] + rt * _ND).astype(jnp.int32).reshape(_NS, c)
        a2 = ei[1].astype(jnp.int32).reshape(_NS, c)
        s2 = jnp.pad(s2, ((0, 0), (0, p - c))).reshape(_NS, nch, _K)
        d2 = jnp.pad(d2, ((0, 0), (0, p - c))).reshape(_NS, nch, _K)
        a2 = jnp.pad(a2, ((0, 0), (0, p - c)),
                     constant_values=_TRASH).reshape(_NS, nch, _K)
        combs.append(jnp.stack([s2, d2, a2], axis=2).reshape(-1, _K))
    comb = jnp.concatenate(combs)
    zrow = jnp.zeros((_ZROWS, _F), f32)
    zden = jnp.zeros((_ZROWS, 16), f32)

    nums, dens = _sc_edge_pass(fs_tbl, el_tbl, er_tbl, comb, zrow, zden)
    nums = nums[:, :_ND]
    dens = dens[:, :_ND]
    num = jnp.stack([nums[0], nums[1], nums[2] + nums[5], nums[3], nums[4]])
    den = jnp.stack([dens[0], dens[1], dens[2] + dens[5], dens[3], dens[4]])

    bs = jnp.stack([b_d2d, b_p2d, b_s2d, b_d2p, b_p2p])
    bp = bs[:, sigma]
    W1p = W1[sigma, :]
    PT = jnp.eye(_F, dtype=f32)[sigma]
    w2v = W2.reshape(1, _F)
    b1v = b1.reshape(1, _F)

    out_d = _semantic_call(num[0:3], den[0:3], bp[0:3],
                           W1p, b1v, w2v, PT, 3)
    out_p = _semantic_call(num[3:5], den[3:5], bp[3:5],
                           W1p, b1v, w2v, PT, 2)
    return (out_d, out_p)
